# baseline (speedup 1.0000x reference)
import functools

import jax
import jax.numpy as jnp
from jax.experimental import pallas as pl
from jax.experimental.pallas import tpu as pltpu

TOPK = 4
GAMMA = 0.1
BB = 128


def _agl_kernel(x_ref, pose_ref, w_ref, b_ref, h_ref, adj_ref,
                *, bb, k, din, dout):
    xb = x_ref[...]
    h3 = jax.lax.dot_general(
        xb, w_ref[...], (((2,), (0,)), ((), ())),
        preferred_element_type=jnp.float32,
    ) + b_ref[...][None]
    h_ref[...] = h3

    sq = jnp.sum(h3 * h3, axis=2)
    g = jax.lax.dot_general(
        h3, h3, (((2,), (2,)), ((0,), (0,))),
        preferred_element_type=jnp.float32,
    )
    d2 = sq[:, :, None] + sq[:, None, :] - 2.0 * g

    work = d2
    for _ in range(TOPK - 1):
        m = jnp.min(work, axis=2, keepdims=True)
        work = jnp.where(work <= m, jnp.float32(jnp.inf), work)
    thresh = jnp.min(work, axis=2, keepdims=True)

    col = jax.lax.broadcasted_iota(jnp.int32, (1, 1, k), 2)
    row = jax.lax.broadcasted_iota(jnp.int32, (1, k, k), 1)
    eye_g = jnp.where(row == col, jnp.float32(GAMMA), jnp.float32(0.0))
    knn_g = jnp.where(d2 <= thresh, jnp.float32(GAMMA), jnp.float32(0.0))
    t = pose_ref[...] + (knn_g + eye_g)
    norm = jnp.sum(t, axis=2, keepdims=True)
    adj_ref[...] = t / norm


@jax.jit
def kernel(x, pose_adj, W, b):
    B, K, DIN = x.shape
    DOUT = W.shape[1]
    grid = (B // BB,)
    b2 = b.reshape(1, DOUT)

    h, adj = pl.pallas_call(
        functools.partial(_agl_kernel, bb=BB, k=K, din=DIN, dout=DOUT),
        grid=grid,
        compiler_params=pltpu.CompilerParams(
            dimension_semantics=("parallel",),
        ),
        in_specs=[
            pl.BlockSpec((BB, K, DIN), lambda i: (i, 0, 0)),
            pl.BlockSpec((BB, K, K), lambda i: (i, 0, 0)),
            pl.BlockSpec((DIN, DOUT), lambda i: (0, 0)),
            pl.BlockSpec((1, DOUT), lambda i: (0, 0)),
        ],
        out_specs=[
            pl.BlockSpec((BB, K, DOUT), lambda i: (i, 0, 0)),
            pl.BlockSpec((BB, K, K), lambda i: (i, 0, 0)),
        ],
        out_shape=[
            jax.ShapeDtypeStruct((B, K, DOUT), jnp.float32),
            jax.ShapeDtypeStruct((B, K, K), jnp.float32),
        ],
    )(x, pose_adj, W, b2)
    return (h, adj)


# speedup vs baseline: 1.0518x; 1.0518x over previous
import functools

import jax
import jax.numpy as jnp
from jax.experimental import pallas as pl
from jax.experimental.pallas import tpu as pltpu

TOPK = 4
GAMMA = 0.1
BB = 256


def _agl_kernel(x_ref, pose_ref, w_ref, b_ref, h_ref, adj_ref,
                *, bb, k, din, dout):
    xb = x_ref[...]
    h3 = jax.lax.dot_general(
        xb, w_ref[...], (((2,), (0,)), ((), ())),
        preferred_element_type=jnp.float32,
    ) + b_ref[...][None]
    h_ref[...] = h3

    sq = jnp.sum(h3 * h3, axis=2)
    g = jax.lax.dot_general(
        h3, h3, (((2,), (2,)), ((0,), (0,))),
        preferred_element_type=jnp.float32,
    )
    d2 = sq[:, :, None] + sq[:, None, :] - 2.0 * g

    col = jax.lax.broadcasted_iota(jnp.int32, (1, 1, k), 2)
    row = jax.lax.broadcasted_iota(jnp.int32, (1, k, k), 1)
    eye_m = row == col
    work = jnp.where(eye_m, jnp.float32(jnp.inf), d2)
    for _ in range(TOPK - 2):
        m = jnp.min(work, axis=2, keepdims=True)
        work = jnp.where(work <= m, jnp.float32(jnp.inf), work)
    thresh = jnp.min(work, axis=2, keepdims=True)

    eye_g = jnp.where(eye_m, jnp.float32(GAMMA), jnp.float32(0.0))
    knn_g = jnp.where(d2 <= thresh, jnp.float32(GAMMA), jnp.float32(0.0))
    t = pose_ref[...] + (knn_g + eye_g)
    norm = jnp.sum(t, axis=2, keepdims=True)
    adj_ref[...] = t / norm


@jax.jit
def kernel(x, pose_adj, W, b):
    B, K, DIN = x.shape
    DOUT = W.shape[1]
    grid = (B // BB,)
    b2 = b.reshape(1, DOUT)

    h, adj = pl.pallas_call(
        functools.partial(_agl_kernel, bb=BB, k=K, din=DIN, dout=DOUT),
        grid=grid,
        compiler_params=pltpu.CompilerParams(
            dimension_semantics=("parallel",),
        ),
        in_specs=[
            pl.BlockSpec((BB, K, DIN), lambda i: (i, 0, 0)),
            pl.BlockSpec((BB, K, K), lambda i: (i, 0, 0)),
            pl.BlockSpec((DIN, DOUT), lambda i: (0, 0)),
            pl.BlockSpec((1, DOUT), lambda i: (0, 0)),
        ],
        out_specs=[
            pl.BlockSpec((BB, K, DOUT), lambda i: (i, 0, 0)),
            pl.BlockSpec((BB, K, K), lambda i: (i, 0, 0)),
        ],
        out_shape=[
            jax.ShapeDtypeStruct((B, K, DOUT), jnp.float32),
            jax.ShapeDtypeStruct((B, K, K), jnp.float32),
        ],
    )(x, pose_adj, W, b2)
    return (h, adj)
